# baseline (speedup 1.0000x reference)
import functools

import jax
import jax.numpy as jnp
from jax.experimental import pallas as pl
from jax.experimental.pallas import tpu as pltpu

R_MAX = 5.0
N_BESSEL = 8
K_NEIGH = 64
EV_TO_KJ_MOL = 96.48533212331002
_BIG = 1e12


def _select_kernel(pos_ref, posT_ref, an_row_ref, d2sel_ref, ansel_ref, *,
                   B, N):
    i = pl.program_id(0)
    dx = pos_ref[pl.ds(i * B, B), 0:1] - posT_ref[0:1, :]
    dy = pos_ref[pl.ds(i * B, B), 1:2] - posT_ref[1:2, :]
    dz = pos_ref[pl.ds(i * B, B), 2:3] - posT_ref[2:3, :]
    d2 = dx * dx + dy * dy + dz * dz
    cols = jax.lax.broadcasted_iota(jnp.int32, (B, N), 1)
    rows = jax.lax.broadcasted_iota(jnp.int32, (B, 1), 0) + i * B
    d2 = jnp.where(cols == rows, _BIG, d2)
    an_row = an_row_ref[:, :]
    code = cols * 16 + an_row
    kcols = jax.lax.broadcasted_iota(jnp.int32, (B, K_NEIGH), 1)

    def body(k, carry):
        d2cur, d2sel, ansel = carry
        mval = jnp.min(d2cur, axis=1, keepdims=True)
        cmin = jnp.min(jnp.where(d2cur == mval, code, jnp.int32(2 ** 30)),
                       axis=1, keepdims=True)
        d2sel = jnp.where(kcols == k, mval, d2sel)
        ansel = jnp.where(kcols == k, jnp.bitwise_and(cmin, 15), ansel)
        jcol = jnp.right_shift(cmin, 4)
        d2cur = jnp.where(cols == jcol, _BIG, d2cur)
        return d2cur, d2sel, ansel

    d2sel0 = jnp.zeros((B, K_NEIGH), jnp.float32)
    ansel0 = jnp.zeros((B, K_NEIGH), jnp.int32)
    _, d2sel, ansel = jax.lax.fori_loop(0, K_NEIGH, body, (d2, d2sel0, ansel0))
    d2sel_ref[:, :] = d2sel
    ansel_ref[:, :] = ansel


def _mlp_kernel(d2_ref, an_ref, anc_ref, W1_ref, b1_ref, W2_ref, b2_ref,
                spe_ref, w_ref, out_ref, *, Ba, F):
    i = pl.program_id(0)
    E = Ba * K_NEIGH
    d2col = d2_ref[:, :]
    d = jnp.sqrt(jnp.maximum(d2col, 0.0) + 1e-12)
    x = d * (1.0 / R_MAX)
    x2 = x * x
    x4 = x2 * x2
    x6 = x4 * x2
    x7 = x6 * x
    x8 = x4 * x4
    fcut = 1.0 - 28.0 * x6 + 48.0 * x7 - 21.0 * x8
    fcut = jnp.where(x < 1.0, fcut, 0.0)
    nvec = (jax.lax.broadcasted_iota(jnp.int32, (1, N_BESSEL), 1)
            .astype(jnp.float32) + 1.0)
    rb = (jnp.sqrt(2.0 / R_MAX) * jnp.sin(nvec * (jnp.pi / R_MAX) * d)
          * (fcut / d))
    hmid = jnp.dot(rb, W1_ref[:, :], precision=jax.lax.Precision.HIGHEST,
                   preferred_element_type=jnp.float32) + b1_ref[:, :]
    hmid = hmid * jax.lax.logistic(hmid)
    Rw = jnp.dot(hmid, W2_ref[:, :], precision=jax.lax.Precision.HIGHEST,
                 preferred_element_type=jnp.float32) + b2_ref[:, :]
    oh_j = (an_ref[:, :] == jax.lax.broadcasted_iota(jnp.int32, (1, 16), 1)
            ).astype(jnp.float32)
    h_j = jnp.dot(oh_j, spe_ref[:, :], precision=jax.lax.Precision.HIGHEST,
                  preferred_element_type=jnp.float32)
    maskf = (d < R_MAX).astype(jnp.float32)
    prod = Rw * h_j * maskf
    m = jnp.sum(prod.reshape(Ba, K_NEIGH, F), axis=1)
    oh_i = (anc_ref[:, :] == jax.lax.broadcasted_iota(jnp.int32, (1, 16), 1)
            ).astype(jnp.float32)
    h_i = jnp.dot(oh_i, spe_ref[:, :], precision=jax.lax.Precision.HIGHEST,
                  preferred_element_type=jnp.float32)
    e_blk = jnp.sum(m * h_i * w_ref[:, :]).reshape(1, 1)

    @pl.when(i == 0)
    def _():
        out_ref[:, :] = jnp.zeros((1, 1), jnp.float32)

    out_ref[:, :] += e_blk


def kernel(positions, atomic_numbers, W1, b1, W2, b2, species_embed, w_read):
    N = positions.shape[0]
    F = species_embed.shape[1]
    B = 128
    Ba = 128
    pos10 = positions.astype(jnp.float32) * 10.0
    pos_pad = jnp.concatenate([pos10, jnp.zeros((N, 5), jnp.float32)], axis=1)
    posT = pos_pad.T.reshape(8, N)
    an = atomic_numbers.astype(jnp.int32)
    an_row = an.reshape(1, N)
    an_col = an.reshape(N, 1)
    spe_pad = jnp.concatenate(
        [species_embed.astype(jnp.float32),
         jnp.zeros((16 - species_embed.shape[0], F), jnp.float32)], axis=0)
    b1r = b1.reshape(1, -1).astype(jnp.float32)
    b2r = b2.reshape(1, -1).astype(jnp.float32)
    wr = w_read.reshape(1, F).astype(jnp.float32)

    d2sel, ansel = pl.pallas_call(
        functools.partial(_select_kernel, B=B, N=N),
        grid=(N // B,),
        in_specs=[pl.BlockSpec(pos_pad.shape, lambda i: (0, 0)),
                  pl.BlockSpec(posT.shape, lambda i: (0, 0)),
                  pl.BlockSpec(an_row.shape, lambda i: (0, 0))],
        out_specs=[pl.BlockSpec((B, K_NEIGH), lambda i: (i, 0)),
                   pl.BlockSpec((B, K_NEIGH), lambda i: (i, 0))],
        out_shape=[jax.ShapeDtypeStruct((N, K_NEIGH), jnp.float32),
                   jax.ShapeDtypeStruct((N, K_NEIGH), jnp.int32)],
        compiler_params=pltpu.CompilerParams(
            dimension_semantics=("parallel",)),
    )(pos_pad, posT, an_row)

    d2_flat = d2sel.reshape(N * K_NEIGH, 1)
    an_flat = ansel.reshape(N * K_NEIGH, 1)
    Eb = Ba * K_NEIGH
    out = pl.pallas_call(
        functools.partial(_mlp_kernel, Ba=Ba, F=F),
        grid=(N // Ba,),
        in_specs=[pl.BlockSpec((Eb, 1), lambda i: (i, 0)),
                  pl.BlockSpec((Eb, 1), lambda i: (i, 0)),
                  pl.BlockSpec((Ba, 1), lambda i: (i, 0)),
                  pl.BlockSpec(W1.shape, lambda i: (0, 0)),
                  pl.BlockSpec((1, b1.shape[0]), lambda i: (0, 0)),
                  pl.BlockSpec(W2.shape, lambda i: (0, 0)),
                  pl.BlockSpec((1, b2.shape[0]), lambda i: (0, 0)),
                  pl.BlockSpec(spe_pad.shape, lambda i: (0, 0)),
                  pl.BlockSpec((1, F), lambda i: (0, 0))],
        out_specs=pl.BlockSpec((1, 1), lambda i: (0, 0)),
        out_shape=jax.ShapeDtypeStruct((1, 1), jnp.float32),
        compiler_params=pltpu.CompilerParams(
            dimension_semantics=("arbitrary",)),
    )(d2_flat, an_flat, an_col, W1, b1r, W2, b2r, spe_pad, wr)
    return out[0, 0] * jnp.float32(EV_TO_KJ_MOL)


# speedup vs baseline: 1.0150x; 1.0150x over previous
import functools

import jax
import jax.numpy as jnp
from jax.experimental import pallas as pl
from jax.experimental.pallas import tpu as pltpu

R_MAX = 5.0
N_BESSEL = 8
K_NEIGH = 64
EV_TO_KJ_MOL = 96.48533212331002
_BIG = 1e12


def _select_kernel(pos_ref, posT_ref, an_row_ref, d2sel_ref, ansel_ref, *,
                   B, N):
    i = pl.program_id(0)
    dx = pos_ref[pl.ds(i * B, B), 0:1] - posT_ref[0:1, :]
    dy = pos_ref[pl.ds(i * B, B), 1:2] - posT_ref[1:2, :]
    dz = pos_ref[pl.ds(i * B, B), 2:3] - posT_ref[2:3, :]
    d2 = dx * dx + dy * dy + dz * dz
    cols = jax.lax.broadcasted_iota(jnp.int32, (B, N), 1)
    rows = jax.lax.broadcasted_iota(jnp.int32, (B, 1), 0) + i * B
    d2 = jnp.where(cols == rows, _BIG, d2)
    an_row = an_row_ref[:, :]
    code = cols * 16 + an_row
    kcols = jax.lax.broadcasted_iota(jnp.int32, (B, K_NEIGH), 1)

    def body(k, carry):
        d2cur, d2sel, ansel = carry
        mval = jnp.min(d2cur, axis=1, keepdims=True)
        cmin = jnp.min(jnp.where(d2cur == mval, code, jnp.int32(2 ** 30)),
                       axis=1, keepdims=True)
        d2sel = jnp.where(kcols == k, mval, d2sel)
        ansel = jnp.where(kcols == k, jnp.bitwise_and(cmin, 15), ansel)
        jcol = jnp.right_shift(cmin, 4)
        d2cur = jnp.where(cols == jcol, _BIG, d2cur)
        return d2cur, d2sel, ansel

    d2sel0 = jnp.zeros((B, K_NEIGH), jnp.float32)
    ansel0 = jnp.zeros((B, K_NEIGH), jnp.int32)
    _, d2sel, ansel = jax.lax.fori_loop(0, K_NEIGH, body, (d2, d2sel0, ansel0))
    d2sel_ref[:, :] = d2sel
    ansel_ref[:, :] = ansel


def _mlp_kernel(d2_ref, an_ref, anc_ref, W1_ref, b1_ref, W2_ref, b2_ref,
                spe_ref, w_ref, out_ref, *, Ba, F):
    i = pl.program_id(0)
    E = Ba * K_NEIGH
    d2col = d2_ref[:, :]
    d = jnp.sqrt(jnp.maximum(d2col, 0.0) + 1e-12)
    x = d * (1.0 / R_MAX)
    x2 = x * x
    x4 = x2 * x2
    x6 = x4 * x2
    x7 = x6 * x
    x8 = x4 * x4
    fcut = 1.0 - 28.0 * x6 + 48.0 * x7 - 21.0 * x8
    fcut = jnp.where(x < 1.0, fcut, 0.0)
    nvec = (jax.lax.broadcasted_iota(jnp.int32, (1, N_BESSEL), 1)
            .astype(jnp.float32) + 1.0)
    rb = (jnp.sqrt(2.0 / R_MAX) * jnp.sin(nvec * (jnp.pi / R_MAX) * d)
          * (fcut / d))
    hmid = jnp.dot(rb, W1_ref[:, :], precision=jax.lax.Precision.HIGHEST,
                   preferred_element_type=jnp.float32) + b1_ref[:, :]
    hmid = hmid * jax.lax.logistic(hmid)
    Rw = jnp.dot(hmid, W2_ref[:, :], precision=jax.lax.Precision.HIGHEST,
                 preferred_element_type=jnp.float32) + b2_ref[:, :]
    oh_j = (an_ref[:, :] == jax.lax.broadcasted_iota(jnp.int32, (1, 16), 1)
            ).astype(jnp.float32)
    h_j = jnp.dot(oh_j, spe_ref[:, :], precision=jax.lax.Precision.HIGHEST,
                  preferred_element_type=jnp.float32)
    maskf = (d < R_MAX).astype(jnp.float32)
    prod = Rw * h_j * maskf
    m = jnp.sum(prod.reshape(Ba, K_NEIGH, F), axis=1)
    oh_i = (anc_ref[:, :] == jax.lax.broadcasted_iota(jnp.int32, (1, 16), 1)
            ).astype(jnp.float32)
    h_i = jnp.dot(oh_i, spe_ref[:, :], precision=jax.lax.Precision.HIGHEST,
                  preferred_element_type=jnp.float32)
    e_blk = jnp.sum(m * h_i * w_ref[:, :]).reshape(1, 1)

    @pl.when(i == 0)
    def _():
        out_ref[:, :] = jnp.zeros((1, 1), jnp.float32)

    out_ref[:, :] += e_blk


def kernel(positions, atomic_numbers, W1, b1, W2, b2, species_embed, w_read):
    N = positions.shape[0]
    F = species_embed.shape[1]
    B = 512
    Ba = 128
    pos10 = positions.astype(jnp.float32) * 10.0
    pos_pad = jnp.concatenate([pos10, jnp.zeros((N, 5), jnp.float32)], axis=1)
    posT = pos_pad.T.reshape(8, N)
    an = atomic_numbers.astype(jnp.int32)
    an_row = an.reshape(1, N)
    an_col = an.reshape(N, 1)
    spe_pad = jnp.concatenate(
        [species_embed.astype(jnp.float32),
         jnp.zeros((16 - species_embed.shape[0], F), jnp.float32)], axis=0)
    b1r = b1.reshape(1, -1).astype(jnp.float32)
    b2r = b2.reshape(1, -1).astype(jnp.float32)
    wr = w_read.reshape(1, F).astype(jnp.float32)

    d2sel, ansel = pl.pallas_call(
        functools.partial(_select_kernel, B=B, N=N),
        grid=(N // B,),
        in_specs=[pl.BlockSpec(pos_pad.shape, lambda i: (0, 0)),
                  pl.BlockSpec(posT.shape, lambda i: (0, 0)),
                  pl.BlockSpec(an_row.shape, lambda i: (0, 0))],
        out_specs=[pl.BlockSpec((B, K_NEIGH), lambda i: (i, 0)),
                   pl.BlockSpec((B, K_NEIGH), lambda i: (i, 0))],
        out_shape=[jax.ShapeDtypeStruct((N, K_NEIGH), jnp.float32),
                   jax.ShapeDtypeStruct((N, K_NEIGH), jnp.int32)],
        compiler_params=pltpu.CompilerParams(
            dimension_semantics=("parallel",)),
    )(pos_pad, posT, an_row)

    d2_flat = d2sel.reshape(N * K_NEIGH, 1)
    an_flat = ansel.reshape(N * K_NEIGH, 1)
    Eb = Ba * K_NEIGH
    out = pl.pallas_call(
        functools.partial(_mlp_kernel, Ba=Ba, F=F),
        grid=(N // Ba,),
        in_specs=[pl.BlockSpec((Eb, 1), lambda i: (i, 0)),
                  pl.BlockSpec((Eb, 1), lambda i: (i, 0)),
                  pl.BlockSpec((Ba, 1), lambda i: (i, 0)),
                  pl.BlockSpec(W1.shape, lambda i: (0, 0)),
                  pl.BlockSpec((1, b1.shape[0]), lambda i: (0, 0)),
                  pl.BlockSpec(W2.shape, lambda i: (0, 0)),
                  pl.BlockSpec((1, b2.shape[0]), lambda i: (0, 0)),
                  pl.BlockSpec(spe_pad.shape, lambda i: (0, 0)),
                  pl.BlockSpec((1, F), lambda i: (0, 0))],
        out_specs=pl.BlockSpec((1, 1), lambda i: (0, 0)),
        out_shape=jax.ShapeDtypeStruct((1, 1), jnp.float32),
        compiler_params=pltpu.CompilerParams(
            dimension_semantics=("arbitrary",)),
    )(d2_flat, an_flat, an_col, W1, b1r, W2, b2r, spe_pad, wr)
    return out[0, 0] * jnp.float32(EV_TO_KJ_MOL)
